# baseline (speedup 1.0000x reference)
import jax
import jax.numpy as jnp
from jax import lax
from jax.experimental import pallas as pl
from jax.experimental.pallas import tpu as pltpu

B, D, V = 128, 512, 100000
CHUNK = 11264
NBLK = (V + CHUNK - 1) // CHUNK
NEG = -1e30


def _body(x_ref, wt_ref, b_ref, act_ref, wv_ref, bv_ref,
          lp_ref, ent_ref, val_ref,
          m_ref, s_ref, t_ref, a_ref):
    i = pl.program_id(0)

    @pl.when(i == 0)
    def _init():
        m_ref[...] = jnp.full((B, 1), NEG, dtype=jnp.float32)
        s_ref[...] = jnp.zeros((B, 1), dtype=jnp.float32)
        t_ref[...] = jnp.zeros((B, 1), dtype=jnp.float32)
        a_ref[...] = jnp.zeros((B, 1), dtype=jnp.float32)
        val_ref[...] = (jnp.sum(x_ref[...] * wv_ref[...], axis=1, keepdims=True)
                        + bv_ref[...])

    L = lax.dot_general(x_ref[...], wt_ref[...],
                        (((1,), (1,)), ((), ())),
                        preferred_element_type=jnp.float32) + b_ref[...]

    cols = jax.lax.broadcasted_iota(jnp.int32, (B, CHUNK), 1) + i * CHUNK
    L = jnp.where(cols < V, L, NEG)

    mc = jnp.max(L, axis=1, keepdims=True)
    m_old = m_ref[...]
    m_new = jnp.maximum(m_old, mc)
    alpha = jnp.exp(m_old - m_new)
    e = jnp.exp(L - m_new)
    s_ref[...] = s_ref[...] * alpha + jnp.sum(e, axis=1, keepdims=True)
    t_ref[...] = t_ref[...] * alpha + jnp.sum(L * e, axis=1, keepdims=True)
    m_ref[...] = m_new

    a_ref[...] += jnp.sum(jnp.where(cols == act_ref[...], L, 0.0),
                          axis=1, keepdims=True)

    @pl.when(i == NBLK - 1)
    def _fin():
        lse = m_ref[...] + jnp.log(s_ref[...])
        lp_ref[...] = a_ref[...] - lse
        ent_ref[...] = lse - t_ref[...] / s_ref[...]


def kernel(x, W_pi, b_pi, W_v, b_v, action):
    act2d = action.astype(jnp.int32).reshape(B, 1)
    b2d = b_pi.reshape(1, V)
    wv2d = W_v.reshape(1, D)
    bv2d = b_v.reshape(1, 1)
    Wt = W_pi.T

    lp, ent, val = pl.pallas_call(
        _body,
        grid=(NBLK,),
        in_specs=[
            pl.BlockSpec((B, D), lambda i: (0, 0)),
            pl.BlockSpec((CHUNK, D), lambda i: (i, 0)),
            pl.BlockSpec((1, CHUNK), lambda i: (0, i)),
            pl.BlockSpec((B, 1), lambda i: (0, 0)),
            pl.BlockSpec((1, D), lambda i: (0, 0)),
            pl.BlockSpec((1, 1), lambda i: (0, 0)),
        ],
        out_specs=[
            pl.BlockSpec((B, 1), lambda i: (0, 0)),
            pl.BlockSpec((B, 1), lambda i: (0, 0)),
            pl.BlockSpec((B, 1), lambda i: (0, 0)),
        ],
        out_shape=[
            jax.ShapeDtypeStruct((B, 1), jnp.float32),
            jax.ShapeDtypeStruct((B, 1), jnp.float32),
            jax.ShapeDtypeStruct((B, 1), jnp.float32),
        ],
        scratch_shapes=[
            pltpu.VMEM((B, 1), jnp.float32),
            pltpu.VMEM((B, 1), jnp.float32),
            pltpu.VMEM((B, 1), jnp.float32),
            pltpu.VMEM((B, 1), jnp.float32),
        ],
        compiler_params=pltpu.CompilerParams(
            dimension_semantics=("arbitrary",),
        ),
    )(x, Wt, b2d, act2d, wv2d, bv2d)

    return (action, lp.reshape(B), ent.reshape(B), val)


# speedup vs baseline: 1.0087x; 1.0087x over previous
import jax
import jax.numpy as jnp
from jax import lax
from jax.experimental import pallas as pl
from jax.experimental.pallas import tpu as pltpu

B, D, V = 128, 512, 100000
CHUNK = 10240
NBLK = (V + CHUNK - 1) // CHUNK
NEG = -1e30


def _body(x_ref, wt_ref, b_ref, act_ref, wv_ref, bv_ref,
          lp_ref, ent_ref, val_ref,
          m_ref, s_ref, t_ref, a_ref):
    i = pl.program_id(0)

    @pl.when(i == 0)
    def _init():
        m_ref[...] = jnp.full((B, 1), NEG, dtype=jnp.float32)
        s_ref[...] = jnp.zeros((B, 1), dtype=jnp.float32)
        t_ref[...] = jnp.zeros((B, 1), dtype=jnp.float32)
        a_ref[...] = jnp.zeros((B, 1), dtype=jnp.float32)
        val_ref[...] = (jnp.sum(x_ref[...] * wv_ref[...], axis=1, keepdims=True)
                        + bv_ref[...])

    L = lax.dot_general(x_ref[...], wt_ref[...],
                        (((1,), (1,)), ((), ())),
                        preferred_element_type=jnp.float32) + b_ref[...]

    cols = jax.lax.broadcasted_iota(jnp.int32, (B, CHUNK), 1) + i * CHUNK
    L = jnp.where(cols < V, L, NEG)

    mc = jnp.max(L, axis=1, keepdims=True)
    m_old = m_ref[...]
    m_new = jnp.maximum(m_old, mc)
    alpha = jnp.exp(m_old - m_new)
    e = jnp.exp(L - m_new)
    s_ref[...] = s_ref[...] * alpha + jnp.sum(e, axis=1, keepdims=True)
    t_ref[...] = t_ref[...] * alpha + jnp.sum(L * e, axis=1, keepdims=True)
    m_ref[...] = m_new

    a_ref[...] += jnp.sum(jnp.where(cols == act_ref[...], L, 0.0),
                          axis=1, keepdims=True)

    @pl.when(i == NBLK - 1)
    def _fin():
        lse = m_ref[...] + jnp.log(s_ref[...])
        lp_ref[...] = a_ref[...] - lse
        ent_ref[...] = lse - t_ref[...] / s_ref[...]


def kernel(x, W_pi, b_pi, W_v, b_v, action):
    act2d = action.astype(jnp.int32).reshape(B, 1)
    b2d = b_pi.reshape(1, V)
    wv2d = W_v.reshape(1, D)
    bv2d = b_v.reshape(1, 1)
    Wt = W_pi.T

    lp, ent, val = pl.pallas_call(
        _body,
        grid=(NBLK,),
        in_specs=[
            pl.BlockSpec((B, D), lambda i: (0, 0)),
            pl.BlockSpec((CHUNK, D), lambda i: (i, 0)),
            pl.BlockSpec((1, CHUNK), lambda i: (0, i)),
            pl.BlockSpec((B, 1), lambda i: (0, 0)),
            pl.BlockSpec((1, D), lambda i: (0, 0)),
            pl.BlockSpec((1, 1), lambda i: (0, 0)),
        ],
        out_specs=[
            pl.BlockSpec((B, 1), lambda i: (0, 0)),
            pl.BlockSpec((B, 1), lambda i: (0, 0)),
            pl.BlockSpec((B, 1), lambda i: (0, 0)),
        ],
        out_shape=[
            jax.ShapeDtypeStruct((B, 1), jnp.float32),
            jax.ShapeDtypeStruct((B, 1), jnp.float32),
            jax.ShapeDtypeStruct((B, 1), jnp.float32),
        ],
        scratch_shapes=[
            pltpu.VMEM((B, 1), jnp.float32),
            pltpu.VMEM((B, 1), jnp.float32),
            pltpu.VMEM((B, 1), jnp.float32),
            pltpu.VMEM((B, 1), jnp.float32),
        ],
        compiler_params=pltpu.CompilerParams(
            dimension_semantics=("arbitrary",),
        ),
    )(x, Wt, b2d, act2d, wv2d, bv2d)

    return (action, lp.reshape(B), ent.reshape(B), val)
